# Initial kernel scaffold; baseline (speedup 1.0000x reference)
import jax
import jax.numpy as jnp
from jax.experimental import pallas as pl


def kernel(x, edge_index, batch, W_in, b_in, W1, b1, W2, b2, W3, b3, Wp1, bp1, Wp2, bp2):
    raise NotImplementedError("write your pallas kernel here")


# speedup vs baseline: 8.8047x; 8.8047x over previous
import jax
import jax.numpy as jnp
from jax import lax
from jax.experimental import pallas as pl
from jax.experimental.pallas import tpu as pltpu
from jax.experimental.pallas import tpu_sc as plsc

N = 10000
E = 160000
F_IN = 256
H = 256
HH = 128
G = 128
P = 128

NT = 16
CE = 128
NCHUNK = 84
EP = NT * CE * NCHUNK
DUMMY = N
NPAD = 10240
WBR = 632
WBL = N - (NT - 1) * WBR

BN = 1000
NB = N // BN

def _mesh():
    return plsc.VectorSubcoreMesh(
        core_axis_name="c", subcore_axis_name="s", num_cores=2, num_subcores=NT
    )


def _sc_deg_body(dstE, dout, slots, didx, dloc, tmp, tmp2):
    cid = lax.axis_index("c")
    tid = lax.axis_index("s")
    zv = jnp.zeros((16,), jnp.float32)

    def _z(i, _):
        dloc[pl.ds(i * 16, 16)] = zv
        return 0

    lax.fori_loop(0, NPAD // 16, _z, 0)
    pltpu.sync_copy(dstE.at[tid], didx)
    ones = jnp.full((16,), 1.0, jnp.float32)
    half = NCHUNK // 2

    def _chunk(j, _):
        jj = cid * half + j

        def _k(k, _):
            idx = didx[jj, pl.ds(k * 16, 16)]
            plsc.addupdate_scatter(dloc, [idx], ones)
            return 0

        lax.fori_loop(0, CE // 16, _k, 0)
        return 0

    lax.fori_loop(0, half, _chunk, 0)
    pltpu.sync_copy(dloc, slots.at[tid])
    plsc.subcore_barrier()

    def _z2(i, _):
        tmp[pl.ds(i * 16, 16)] = zv
        return 0

    lax.fori_loop(0, 640 // 16, _z2, 0)

    def _s(s, _):
        pltpu.sync_copy(slots.at[s, pl.ds(pl.multiple_of(tid * 640, 128), 640)], tmp2)

        def _a(i, _):
            tmp[pl.ds(i * 16, 16)] = tmp[pl.ds(i * 16, 16)] + tmp2[pl.ds(i * 16, 16)]
            return 0

        lax.fori_loop(0, 640 // 16, _a, 0)
        return 0

    lax.fori_loop(0, NT, _s, 0)

    obase = pl.multiple_of(tid * 640, 128)

    @pl.when(cid == 0)
    def _():
        pltpu.sync_copy(tmp, dout.at[0, pl.ds(obase, 640)])

    @pl.when(cid == 1)
    def _():
        pltpu.sync_copy(tmp, dout.at[1, pl.ds(obase, 640)])


def _deg_call(dstE3):
    return pl.kernel(
        _sc_deg_body,
        out_type=jax.ShapeDtypeStruct((2, NPAD), jnp.float32),
        mesh=_mesh(),
        compiler_params=pltpu.CompilerParams(needs_layout_passes=False),
        scratch_types=[
            pltpu.VMEM_SHARED((NT, NPAD), jnp.float32),
            pltpu.VMEM((NCHUNK, CE), jnp.int32),
            pltpu.VMEM((NPAD,), jnp.float32),
            pltpu.VMEM((640,), jnp.float32),
            pltpu.VMEM((640,), jnp.float32),
        ],
    )(dstE3)


def _sc_layer_body(gA, gB, srcE, dstE, outA, outB, acc, sidx, didx, gbuf, sem):
    cid = lax.axis_index("c")
    tid = lax.axis_index("s")
    zv = jnp.zeros((16,), jnp.float32)

    def _zr(r, _):
        def _zc(k, _):
            gbuf[r, pl.ds(k * 16, 16)] = zv
            return 0

        lax.fori_loop(0, HH // 16, _zc, 0)
        return 0

    lax.fori_loop(0, CE, _zr, 0)

    def _za(k, _):
        pltpu.sync_copy(gbuf, acc.at[pl.ds(pl.multiple_of(tid * 640 + k * 128, 128), 128)])
        return 0

    lax.fori_loop(0, 5, _za, 0)
    pltpu.sync_copy(srcE.at[tid], sidx)
    pltpu.sync_copy(dstE.at[tid], didx)
    plsc.subcore_barrier()

    def _run(g, out):
        def _chunk(j, _):
            pltpu.async_copy(g.at[sidx.at[j]], gbuf, sem).wait()
            pltpu.sync_copy(gbuf, acc.at[didx.at[j]], add=True)
            return 0

        lax.fori_loop(0, NCHUNK, _chunk, 0)
        plsc.subcore_barrier()
        base = pl.multiple_of(tid * WBR, 8)

        @pl.when(tid < NT - 1)
        def _():
            pltpu.sync_copy(acc.at[pl.ds(base, WBR)], out.at[pl.ds(base, WBR)])

        @pl.when(tid == NT - 1)
        def _():
            pltpu.sync_copy(acc.at[pl.ds(base, WBL)], out.at[pl.ds(base, WBL)])

    @pl.when(cid == 0)
    def _():
        _run(gA, outA)

    @pl.when(cid == 1)
    def _():
        _run(gB, outB)


def _layer_call(gA, gB, srcE3, dstE3):
    return pl.kernel(
        _sc_layer_body,
        out_type=(
            jax.ShapeDtypeStruct((N, HH), jnp.float32),
            jax.ShapeDtypeStruct((N, HH), jnp.float32),
        ),
        mesh=_mesh(),
        scratch_types=[
            pltpu.VMEM_SHARED((NPAD, HH), jnp.float32),
            pltpu.VMEM((NCHUNK, CE), jnp.int32),
            pltpu.VMEM((NCHUNK, CE), jnp.int32),
            pltpu.VMEM((CE, HH), jnp.float32),
            pltpu.SemaphoreType.DMA,
        ],
    )(gA, gB, srcE3, dstE3)


def _k0_body(x_ref, win_ref, bin_ref, w1_ref, d0_ref, d1_ref, gA_ref, gB_ref, dinv_ref):
    h = jnp.dot(x_ref[...], win_ref[...], preferred_element_type=jnp.float32)
    h = h + bin_ref[...]
    g = jnp.dot(h, w1_ref[...], preferred_element_type=jnp.float32)
    deg = d0_ref[0, 0, :] + d1_ref[0, 0, :]
    dinv = lax.rsqrt(deg)
    dinv_ref[0, 0, :] = dinv
    gs = g * dinv[:, None]
    gA_ref[...] = gs[:, :HH]
    gB_ref[...] = gs[:, HH:]


def _k12_body(aA_ref, aB_ref, dinv_ref, b_ref, w_ref, x_ref, gA_ref, gB_ref):
    dinv = dinv_ref[0, 0, :]
    a = jnp.concatenate([aA_ref[...], aB_ref[...]], axis=1)
    xl = jnp.maximum(a * dinv[:, None] + b_ref[...], 0.0)
    x_ref[...] = xl
    g = jnp.dot(xl, w_ref[...], preferred_element_type=jnp.float32) * dinv[:, None]
    gA_ref[...] = g[:, :HH]
    gB_ref[...] = g[:, HH:]


def _k3_body(aA_ref, aB_ref, dinv_ref, b3_ref, x1_ref, x2_ref, batch_ref,
             wp1_ref, bp1_ref, wp2_ref, bp2_ref, z_ref, pooled, cnt):
    i = pl.program_id(0)

    @pl.when(i == 0)
    def _():
        pooled[...] = jnp.zeros_like(pooled)
        cnt[...] = jnp.zeros_like(cnt)

    dinv = dinv_ref[0, 0, :]
    a = jnp.concatenate([aA_ref[...], aB_ref[...]], axis=1)
    x3 = jnp.maximum(a * dinv[:, None] + b3_ref[...], 0.0)
    xs = jnp.concatenate([x1_ref[...], x2_ref[...], x3], axis=1)
    b = batch_ref[0, 0, :]
    gid = lax.broadcasted_iota(jnp.int32, (G, BN), 0)
    mask = (gid == b[None, :]).astype(jnp.float32)
    pooled[...] += jnp.dot(mask, xs, preferred_element_type=jnp.float32)
    cnt[...] += jnp.sum(mask, axis=1)[:, None]

    @pl.when(i == pl.num_programs(0) - 1)
    def _():
        mean = pooled[...] / jnp.maximum(cnt[...], 1.0)
        z1 = jnp.dot(mean, wp1_ref[...], preferred_element_type=jnp.float32)
        z1 = jnp.maximum(z1 + bp1_ref[...], 0.0)
        z = jnp.dot(z1, wp2_ref[...], preferred_element_type=jnp.float32) + bp2_ref[...]
        nrm = jnp.sqrt(jnp.sum(z * z, axis=1, keepdims=True))
        z_ref[...] = z / jnp.maximum(nrm, 1e-12)


def _row_spec(cols):
    return pl.BlockSpec((BN, cols), lambda i: (i, 0))


def _full_spec(shape):
    nd = len(shape)
    return pl.BlockSpec(shape, lambda i: (0,) * nd)


_VEC_SPEC = pl.BlockSpec((1, 1, BN), lambda i: (i, 0, 0))


def _k0(x, Win, b_in, W1, d0, d1):
    return pl.pallas_call(
        _k0_body,
        grid=(NB,),
        in_specs=[
            _row_spec(F_IN),
            _full_spec((F_IN, H)),
            _full_spec((1, H)),
            _full_spec((H, H)),
            _VEC_SPEC,
            _VEC_SPEC,
        ],
        out_specs=[_row_spec(HH), _row_spec(HH), _VEC_SPEC],
        out_shape=[
            jax.ShapeDtypeStruct((N, HH), jnp.float32),
            jax.ShapeDtypeStruct((N, HH), jnp.float32),
            jax.ShapeDtypeStruct((NB, 1, BN), jnp.float32),
        ],
    )(x, Win, b_in, W1, d0, d1)


def _k12(aA, aB, dinv, b, W):
    return pl.pallas_call(
        _k12_body,
        grid=(NB,),
        in_specs=[
            _row_spec(HH),
            _row_spec(HH),
            _VEC_SPEC,
            _full_spec((1, H)),
            _full_spec((H, H)),
        ],
        out_specs=[_row_spec(H), _row_spec(HH), _row_spec(HH)],
        out_shape=[
            jax.ShapeDtypeStruct((N, H), jnp.float32),
            jax.ShapeDtypeStruct((N, HH), jnp.float32),
            jax.ShapeDtypeStruct((N, HH), jnp.float32),
        ],
    )(aA, aB, dinv, b, W)


def _k3(aA, aB, dinv, b3, x1, x2, batch3, Wp1, bp1, Wp2, bp2):
    return pl.pallas_call(
        _k3_body,
        grid=(NB,),
        in_specs=[
            _row_spec(HH),
            _row_spec(HH),
            _VEC_SPEC,
            _full_spec((1, H)),
            _row_spec(H),
            _row_spec(H),
            pl.BlockSpec((1, 1, BN), lambda i: (i, 0, 0)),
            _full_spec((3 * H, H)),
            _full_spec((1, H)),
            _full_spec((H, P)),
            _full_spec((1, P)),
        ],
        out_specs=pl.BlockSpec((G, P), lambda i: (0, 0)),
        out_shape=jax.ShapeDtypeStruct((G, P), jnp.float32),
        scratch_shapes=[
            pltpu.VMEM((G, 3 * H), jnp.float32),
            pltpu.VMEM((G, 1), jnp.float32),
        ],
    )(aA, aB, dinv, b3, x1, x2, batch3, Wp1, bp1, Wp2, bp2)


def kernel(x, edge_index, batch, W_in, b_in, W1, b1, W2, b2, W3, b3, Wp1, bp1, Wp2, bp2):
    src = edge_index[0]
    dst = edge_index[1]
    loop = jnp.arange(N, dtype=jnp.int32)
    pad = EP - E - N
    srcE = jnp.concatenate([src, loop, jnp.zeros((pad,), jnp.int32)])
    dstE = jnp.concatenate([dst, loop, jnp.full((pad,), DUMMY, jnp.int32)])
    srcE3 = srcE.reshape(NT, NCHUNK, CE)
    dstE3 = dstE.reshape(NT, NCHUNK, CE)

    deg2 = _deg_call(dstE3)
    d0 = deg2[0, :N].reshape(NB, 1, BN)
    d1 = deg2[1, :N].reshape(NB, 1, BN)

    b_in2 = b_in.reshape(1, H)
    b1_2 = b1.reshape(1, H)
    b2_2 = b2.reshape(1, H)
    b3_2 = b3.reshape(1, H)
    bp1_2 = bp1.reshape(1, H)
    bp2_2 = bp2.reshape(1, P)
    batch3 = batch.reshape(NB, 1, BN)

    gA, gB, dinv = _k0(x, W_in, b_in2, W1, d0, d1)
    a1A, a1B = _layer_call(gA, gB, srcE3, dstE3)
    x1, g2A, g2B = _k12(a1A, a1B, dinv, b1_2, W2)
    a2A, a2B = _layer_call(g2A, g2B, srcE3, dstE3)
    x2, g3A, g3B = _k12(a2A, a2B, dinv, b2_2, W3)
    a3A, a3B = _layer_call(g3A, g3B, srcE3, dstE3)
    z = _k3(a3A, a3B, dinv, b3_2, x1, x2, batch3, Wp1, bp1_2, Wp2, bp2_2)
    return z


# speedup vs baseline: 10.7681x; 1.2230x over previous
import jax
import jax.numpy as jnp
from jax import lax
from jax.experimental import pallas as pl
from jax.experimental.pallas import tpu as pltpu
from jax.experimental.pallas import tpu_sc as plsc

N = 10000
E = 160000
F_IN = 256
H = 256
HH = 128
G = 128
P = 128

NT = 16
CE = 128
NCHUNK = 84
EP = NT * CE * NCHUNK
DUMMY = N
NPAD = 10240
WBR = 632
WBL = N - (NT - 1) * WBR

BN = 1000
NB = N // BN

def _mesh():
    return plsc.VectorSubcoreMesh(
        core_axis_name="c", subcore_axis_name="s", num_cores=2, num_subcores=NT
    )


def _sc_deg_body(dstE, dout, slots, didx, dloc, tmp, tmp2):
    cid = lax.axis_index("c")
    tid = lax.axis_index("s")
    zv = jnp.zeros((16,), jnp.float32)

    def _z(i, _):
        dloc[pl.ds(i * 16, 16)] = zv
        return 0

    lax.fori_loop(0, NPAD // 16, _z, 0)
    pltpu.sync_copy(dstE.at[tid], didx)
    ones = jnp.full((16,), 1.0, jnp.float32)
    half = NCHUNK // 2

    def _chunk(j, _):
        jj = cid * half + j

        def _k(k, _):
            idx = didx[jj, pl.ds(k * 16, 16)]
            plsc.addupdate_scatter(dloc, [idx], ones)
            return 0

        lax.fori_loop(0, CE // 16, _k, 0)
        return 0

    lax.fori_loop(0, half, _chunk, 0)
    pltpu.sync_copy(dloc, slots.at[tid])
    plsc.subcore_barrier()

    def _z2(i, _):
        tmp[pl.ds(i * 16, 16)] = zv
        return 0

    lax.fori_loop(0, 640 // 16, _z2, 0)

    def _s(s, _):
        pltpu.sync_copy(slots.at[s, pl.ds(pl.multiple_of(tid * 640, 128), 640)], tmp2)

        def _a(i, _):
            tmp[pl.ds(i * 16, 16)] = tmp[pl.ds(i * 16, 16)] + tmp2[pl.ds(i * 16, 16)]
            return 0

        lax.fori_loop(0, 640 // 16, _a, 0)
        return 0

    lax.fori_loop(0, NT, _s, 0)

    obase = pl.multiple_of(tid * 640, 128)

    @pl.when(cid == 0)
    def _():
        pltpu.sync_copy(tmp, dout.at[0, pl.ds(obase, 640)])

    @pl.when(cid == 1)
    def _():
        pltpu.sync_copy(tmp, dout.at[1, pl.ds(obase, 640)])


def _deg_call(dstE3):
    return pl.kernel(
        _sc_deg_body,
        out_type=jax.ShapeDtypeStruct((2, NPAD), jnp.float32),
        mesh=_mesh(),
        compiler_params=pltpu.CompilerParams(needs_layout_passes=False),
        scratch_types=[
            pltpu.VMEM_SHARED((NT, NPAD), jnp.float32),
            pltpu.VMEM((NCHUNK, CE), jnp.int32),
            pltpu.VMEM((NPAD,), jnp.float32),
            pltpu.VMEM((640,), jnp.float32),
            pltpu.VMEM((640,), jnp.float32),
        ],
    )(dstE3)


NCH2 = NCHUNK // 2


def _sc_layer_body(gA, gB, srcP, dstP, outA, outB,
                   acc, spk, dpk, sb0, sb1, db, gbuf0, gbuf1, sem0, sem1):
    cid = lax.axis_index("c")
    tid = lax.axis_index("s")
    zv = jnp.zeros((16,), jnp.float32)

    def _zr(r, _):
        def _zc(k, _):
            gbuf0[r, pl.ds(k * 16, 16)] = zv
            return 0

        lax.fori_loop(0, HH // 16, _zc, 0)
        return 0

    lax.fori_loop(0, CE, _zr, 0)

    def _za(k, _):
        pltpu.sync_copy(gbuf0, acc.at[pl.ds(pl.multiple_of(tid * 640 + k * 128, 128), 128)])
        return 0

    lax.fori_loop(0, 5, _za, 0)
    pltpu.sync_copy(srcP.at[tid], spk)
    pltpu.sync_copy(dstP.at[tid], dpk)
    plsc.subcore_barrier()

    def _unpack(pk, m, off, buf):
        for k in range(4):
            w = pk[m, pl.ds(off + k * 16, 16)]
            buf[pl.ds(k * 16, 16)] = w & 0xFFFF
            buf[pl.ds(64 + k * 16, 16)] = lax.shift_right_logical(w, 16)

    def _run(g, out):
        _unpack(spk, 0, 0, sb0)
        pltpu.async_copy(g.at[sb0], gbuf0, sem0)

        def _pair(j2, _):
            _unpack(spk, j2, 64, sb1)
            pltpu.async_copy(g.at[sb1], gbuf1, sem1)
            _unpack(dpk, j2, 0, db)
            pltpu.make_async_copy(g.at[sb0], gbuf0, sem0).wait()
            pltpu.sync_copy(gbuf0, acc.at[db], add=True)

            @pl.when(j2 < NCH2 - 1)
            def _():
                _unpack(spk, j2 + 1, 0, sb0)
                pltpu.async_copy(g.at[sb0], gbuf0, sem0)

            _unpack(dpk, j2, 64, db)
            pltpu.make_async_copy(g.at[sb1], gbuf1, sem1).wait()
            pltpu.sync_copy(gbuf1, acc.at[db], add=True)
            return 0

        lax.fori_loop(0, NCH2, _pair, 0)
        plsc.subcore_barrier()
        base = pl.multiple_of(tid * WBR, 8)

        @pl.when(tid < NT - 1)
        def _():
            pltpu.sync_copy(acc.at[pl.ds(base, WBR)], out.at[pl.ds(base, WBR)])

        @pl.when(tid == NT - 1)
        def _():
            pltpu.sync_copy(acc.at[pl.ds(base, WBL)], out.at[pl.ds(base, WBL)])

    @pl.when(cid == 0)
    def _():
        _run(gA, outA)

    @pl.when(cid == 1)
    def _():
        _run(gB, outB)


def _layer_call(gA, gB, srcP, dstP):
    return pl.kernel(
        _sc_layer_body,
        out_type=(
            jax.ShapeDtypeStruct((N, HH), jnp.float32),
            jax.ShapeDtypeStruct((N, HH), jnp.float32),
        ),
        mesh=_mesh(),
        scratch_types=[
            pltpu.VMEM_SHARED((NPAD, HH), jnp.float32),
            pltpu.VMEM((NCH2, CE), jnp.int32),
            pltpu.VMEM((NCH2, CE), jnp.int32),
            pltpu.VMEM((CE,), jnp.int32),
            pltpu.VMEM((CE,), jnp.int32),
            pltpu.VMEM((CE,), jnp.int32),
            pltpu.VMEM((CE, HH), jnp.float32),
            pltpu.VMEM((CE, HH), jnp.float32),
            pltpu.SemaphoreType.DMA,
            pltpu.SemaphoreType.DMA,
        ],
    )(gA, gB, srcP, dstP)


def _k0_body(x_ref, win_ref, bin_ref, w1_ref, d0_ref, d1_ref, gA_ref, gB_ref, dinv_ref):
    h = jnp.dot(x_ref[...], win_ref[...], preferred_element_type=jnp.float32)
    h = h + bin_ref[...]
    g = jnp.dot(h, w1_ref[...], preferred_element_type=jnp.float32)
    deg = d0_ref[0, 0, :] + d1_ref[0, 0, :]
    dinv = lax.rsqrt(deg)
    dinv_ref[0, 0, :] = dinv
    gs = g * dinv[:, None]
    gA_ref[...] = gs[:, :HH]
    gB_ref[...] = gs[:, HH:]


def _k12_body(aA_ref, aB_ref, dinv_ref, b_ref, w_ref, x_ref, gA_ref, gB_ref):
    dinv = dinv_ref[0, 0, :]
    a = jnp.concatenate([aA_ref[...], aB_ref[...]], axis=1)
    xl = jnp.maximum(a * dinv[:, None] + b_ref[...], 0.0)
    x_ref[...] = xl
    g = jnp.dot(xl, w_ref[...], preferred_element_type=jnp.float32) * dinv[:, None]
    gA_ref[...] = g[:, :HH]
    gB_ref[...] = g[:, HH:]


def _k3_body(aA_ref, aB_ref, dinv_ref, b3_ref, x1_ref, x2_ref, batch_ref,
             wp1_ref, bp1_ref, wp2_ref, bp2_ref, z_ref, pooled, cnt):
    i = pl.program_id(0)

    @pl.when(i == 0)
    def _():
        pooled[...] = jnp.zeros_like(pooled)
        cnt[...] = jnp.zeros_like(cnt)

    dinv = dinv_ref[0, 0, :]
    a = jnp.concatenate([aA_ref[...], aB_ref[...]], axis=1)
    x3 = jnp.maximum(a * dinv[:, None] + b3_ref[...], 0.0)
    xs = jnp.concatenate([x1_ref[...], x2_ref[...], x3], axis=1)
    b = batch_ref[0, 0, :]
    gid = lax.broadcasted_iota(jnp.int32, (G, BN), 0)
    mask = (gid == b[None, :]).astype(jnp.float32)
    pooled[...] += jnp.dot(mask, xs, preferred_element_type=jnp.float32)
    cnt[...] += jnp.sum(mask, axis=1)[:, None]

    @pl.when(i == pl.num_programs(0) - 1)
    def _():
        mean = pooled[...] / jnp.maximum(cnt[...], 1.0)
        z1 = jnp.dot(mean, wp1_ref[...], preferred_element_type=jnp.float32)
        z1 = jnp.maximum(z1 + bp1_ref[...], 0.0)
        z = jnp.dot(z1, wp2_ref[...], preferred_element_type=jnp.float32) + bp2_ref[...]
        nrm = jnp.sqrt(jnp.sum(z * z, axis=1, keepdims=True))
        z_ref[...] = z / jnp.maximum(nrm, 1e-12)


def _row_spec(cols):
    return pl.BlockSpec((BN, cols), lambda i: (i, 0))


def _full_spec(shape):
    nd = len(shape)
    return pl.BlockSpec(shape, lambda i: (0,) * nd)


_VEC_SPEC = pl.BlockSpec((1, 1, BN), lambda i: (i, 0, 0))


def _k0(x, Win, b_in, W1, d0, d1):
    return pl.pallas_call(
        _k0_body,
        grid=(NB,),
        in_specs=[
            _row_spec(F_IN),
            _full_spec((F_IN, H)),
            _full_spec((1, H)),
            _full_spec((H, H)),
            _VEC_SPEC,
            _VEC_SPEC,
        ],
        out_specs=[_row_spec(HH), _row_spec(HH), _VEC_SPEC],
        out_shape=[
            jax.ShapeDtypeStruct((N, HH), jnp.float32),
            jax.ShapeDtypeStruct((N, HH), jnp.float32),
            jax.ShapeDtypeStruct((NB, 1, BN), jnp.float32),
        ],
    )(x, Win, b_in, W1, d0, d1)


def _k12(aA, aB, dinv, b, W):
    return pl.pallas_call(
        _k12_body,
        grid=(NB,),
        in_specs=[
            _row_spec(HH),
            _row_spec(HH),
            _VEC_SPEC,
            _full_spec((1, H)),
            _full_spec((H, H)),
        ],
        out_specs=[_row_spec(H), _row_spec(HH), _row_spec(HH)],
        out_shape=[
            jax.ShapeDtypeStruct((N, H), jnp.float32),
            jax.ShapeDtypeStruct((N, HH), jnp.float32),
            jax.ShapeDtypeStruct((N, HH), jnp.float32),
        ],
    )(aA, aB, dinv, b, W)


def _k3(aA, aB, dinv, b3, x1, x2, batch3, Wp1, bp1, Wp2, bp2):
    return pl.pallas_call(
        _k3_body,
        grid=(NB,),
        in_specs=[
            _row_spec(HH),
            _row_spec(HH),
            _VEC_SPEC,
            _full_spec((1, H)),
            _row_spec(H),
            _row_spec(H),
            pl.BlockSpec((1, 1, BN), lambda i: (i, 0, 0)),
            _full_spec((3 * H, H)),
            _full_spec((1, H)),
            _full_spec((H, P)),
            _full_spec((1, P)),
        ],
        out_specs=pl.BlockSpec((G, P), lambda i: (0, 0)),
        out_shape=jax.ShapeDtypeStruct((G, P), jnp.float32),
        scratch_shapes=[
            pltpu.VMEM((G, 3 * H), jnp.float32),
            pltpu.VMEM((G, 1), jnp.float32),
        ],
    )(aA, aB, dinv, b3, x1, x2, batch3, Wp1, bp1, Wp2, bp2)


def kernel(x, edge_index, batch, W_in, b_in, W1, b1, W2, b2, W3, b3, Wp1, bp1, Wp2, bp2):
    src = edge_index[0]
    dst = edge_index[1]
    loop = jnp.arange(N, dtype=jnp.int32)
    pad = EP - E - N
    srcE = jnp.concatenate([src, loop, jnp.zeros((pad,), jnp.int32)])
    dstE = jnp.concatenate([dst, loop, jnp.full((pad,), DUMMY, jnp.int32)])
    dstE3 = dstE.reshape(NT, NCHUNK, CE)
    s5 = srcE.reshape(NT, NCH2, 2, 2, CE // 2)
    d5 = dstE.reshape(NT, NCH2, 2, 2, CE // 2)
    srcP = (s5[:, :, :, 0, :] | (s5[:, :, :, 1, :] << 16)).reshape(NT, NCH2, CE)
    dstP = (d5[:, :, :, 0, :] | (d5[:, :, :, 1, :] << 16)).reshape(NT, NCH2, CE)

    deg2 = _deg_call(dstE3)
    d0 = deg2[0, :N].reshape(NB, 1, BN)
    d1 = deg2[1, :N].reshape(NB, 1, BN)

    b_in2 = b_in.reshape(1, H)
    b1_2 = b1.reshape(1, H)
    b2_2 = b2.reshape(1, H)
    b3_2 = b3.reshape(1, H)
    bp1_2 = bp1.reshape(1, H)
    bp2_2 = bp2.reshape(1, P)
    batch3 = batch.reshape(NB, 1, BN)

    gA, gB, dinv = _k0(x, W_in, b_in2, W1, d0, d1)
    a1A, a1B = _layer_call(gA, gB, srcP, dstP)
    x1, g2A, g2B = _k12(a1A, a1B, dinv, b1_2, W2)
    a2A, a2B = _layer_call(g2A, g2B, srcP, dstP)
    x2, g3A, g3B = _k12(a2A, a2B, dinv, b2_2, W3)
    a3A, a3B = _layer_call(g3A, g3B, srcP, dstP)
    z = _k3(a3A, a3B, dinv, b3_2, x1, x2, batch3, Wp1, bp1_2, Wp2, bp2_2)
    return z


# speedup vs baseline: 10.8248x; 1.0053x over previous
import jax
import jax.numpy as jnp
from jax import lax
from jax.experimental import pallas as pl
from jax.experimental.pallas import tpu as pltpu
from jax.experimental.pallas import tpu_sc as plsc

N = 10000
E = 160000
F_IN = 256
H = 256
HH = 128
G = 128
P = 128

NT = 16
CE = 128
NCHUNK = 84
EP = NT * CE * NCHUNK
DUMMY = N
NPAD = 10240
WBR = 632
WBL = N - (NT - 1) * WBR

BN = 1000
NB = N // BN

def _mesh():
    return plsc.VectorSubcoreMesh(
        core_axis_name="c", subcore_axis_name="s", num_cores=2, num_subcores=NT
    )


def _sc_deg_body(dstE, dout, slots, didx, dloc, tmp, tmp2):
    cid = lax.axis_index("c")
    tid = lax.axis_index("s")
    zv = jnp.zeros((16,), jnp.float32)

    def _z(i, _):
        dloc[pl.ds(i * 16, 16)] = zv
        return 0

    lax.fori_loop(0, NPAD // 16, _z, 0)
    pltpu.sync_copy(dstE.at[tid], didx)
    ones = jnp.full((16,), 1.0, jnp.float32)
    half = NCHUNK // 2

    def _chunk(j, _):
        jj = cid * half + j

        def _k(k, _):
            idx = didx[jj, pl.ds(k * 16, 16)]
            plsc.addupdate_scatter(dloc, [idx], ones)
            return 0

        lax.fori_loop(0, CE // 16, _k, 0)
        return 0

    lax.fori_loop(0, half, _chunk, 0)
    pltpu.sync_copy(dloc, slots.at[tid])
    plsc.subcore_barrier()

    def _z2(i, _):
        tmp[pl.ds(i * 16, 16)] = zv
        return 0

    lax.fori_loop(0, 640 // 16, _z2, 0)

    def _s(s, _):
        pltpu.sync_copy(slots.at[s, pl.ds(pl.multiple_of(tid * 640, 128), 640)], tmp2)

        def _a(i, _):
            tmp[pl.ds(i * 16, 16)] = tmp[pl.ds(i * 16, 16)] + tmp2[pl.ds(i * 16, 16)]
            return 0

        lax.fori_loop(0, 640 // 16, _a, 0)
        return 0

    lax.fori_loop(0, NT, _s, 0)

    obase = pl.multiple_of(tid * 640, 128)

    @pl.when(cid == 0)
    def _():
        pltpu.sync_copy(tmp, dout.at[0, pl.ds(obase, 640)])

    @pl.when(cid == 1)
    def _():
        pltpu.sync_copy(tmp, dout.at[1, pl.ds(obase, 640)])


def _deg_call(dstE3):
    return pl.kernel(
        _sc_deg_body,
        out_type=jax.ShapeDtypeStruct((2, NPAD), jnp.float32),
        mesh=_mesh(),
        compiler_params=pltpu.CompilerParams(needs_layout_passes=False),
        scratch_types=[
            pltpu.VMEM_SHARED((NT, NPAD), jnp.float32),
            pltpu.VMEM((NCHUNK, CE), jnp.int32),
            pltpu.VMEM((NPAD,), jnp.float32),
            pltpu.VMEM((640,), jnp.float32),
            pltpu.VMEM((640,), jnp.float32),
        ],
    )(dstE3)


NCH2 = NCHUNK // 2


def _sc_layer_body(g, srcP, dstP, out,
                   acc, spk, dpk, sb0, sb1, db, gbuf0, gbuf1, sem0, sem1):
    cid = lax.axis_index("c")
    tid = lax.axis_index("s")
    zv = jnp.zeros((16,), jnp.float32)
    goff = jnp.full((16,), 0, jnp.int32) + cid * N

    def _zr(r, _):
        def _zc(k, _):
            gbuf0[r, pl.ds(k * 16, 16)] = zv
            return 0

        lax.fori_loop(0, HH // 16, _zc, 0)
        return 0

    lax.fori_loop(0, CE, _zr, 0)

    def _za(k, _):
        pltpu.sync_copy(gbuf0, acc.at[pl.ds(pl.multiple_of(tid * 640 + k * 128, 128), 128)])
        return 0

    lax.fori_loop(0, 5, _za, 0)
    pltpu.sync_copy(srcP.at[tid], spk)
    pltpu.sync_copy(dstP.at[tid], dpk)
    plsc.subcore_barrier()

    def _unpack_src(m, off, buf):
        for k in range(4):
            w = spk[m, pl.ds(off + k * 16, 16)]
            buf[pl.ds(k * 16, 16)] = (w & 0xFFFF) + goff
            buf[pl.ds(64 + k * 16, 16)] = lax.shift_right_logical(w, 16) + goff

    def _unpack_dst(m, off, buf):
        for k in range(4):
            w = dpk[m, pl.ds(off + k * 16, 16)]
            buf[pl.ds(k * 16, 16)] = w & 0xFFFF
            buf[pl.ds(64 + k * 16, 16)] = lax.shift_right_logical(w, 16)

    _unpack_src(0, 0, sb0)
    pltpu.async_copy(g.at[sb0], gbuf0, sem0)

    def _pair(j2, _):
        _unpack_src(j2, 64, sb1)
        pltpu.async_copy(g.at[sb1], gbuf1, sem1)
        _unpack_dst(j2, 0, db)
        pltpu.make_async_copy(g.at[sb0], gbuf0, sem0).wait()
        pltpu.sync_copy(gbuf0, acc.at[db], add=True)

        @pl.when(j2 < NCH2 - 1)
        def _():
            _unpack_src(j2 + 1, 0, sb0)
            pltpu.async_copy(g.at[sb0], gbuf0, sem0)

        _unpack_dst(j2, 64, db)
        pltpu.make_async_copy(g.at[sb1], gbuf1, sem1).wait()
        pltpu.sync_copy(gbuf1, acc.at[db], add=True)
        return 0

    lax.fori_loop(0, NCH2, _pair, 0)
    plsc.subcore_barrier()
    base = pl.multiple_of(tid * WBR, 8)
    obase = pl.multiple_of(cid * N + tid * WBR, 8)

    @pl.when(tid < NT - 1)
    def _():
        pltpu.sync_copy(acc.at[pl.ds(base, WBR)], out.at[pl.ds(obase, WBR)])

    @pl.when(tid == NT - 1)
    def _():
        pltpu.sync_copy(acc.at[pl.ds(base, WBL)], out.at[pl.ds(obase, WBL)])


def _layer_call(g2, srcP, dstP):
    return pl.kernel(
        _sc_layer_body,
        out_type=jax.ShapeDtypeStruct((2 * N, HH), jnp.float32),
        mesh=_mesh(),
        scratch_types=[
            pltpu.VMEM_SHARED((NPAD, HH), jnp.float32),
            pltpu.VMEM((NCH2, CE), jnp.int32),
            pltpu.VMEM((NCH2, CE), jnp.int32),
            pltpu.VMEM((CE,), jnp.int32),
            pltpu.VMEM((CE,), jnp.int32),
            pltpu.VMEM((CE,), jnp.int32),
            pltpu.VMEM((CE, HH), jnp.float32),
            pltpu.VMEM((CE, HH), jnp.float32),
            pltpu.SemaphoreType.DMA,
            pltpu.SemaphoreType.DMA,
        ],
    )(g2, srcP, dstP)


def _k0a_body(x_ref, win_ref, bin_ref, w1_ref, u_ref):
    h = jnp.dot(x_ref[...], win_ref[...], preferred_element_type=jnp.float32)
    h = h + bin_ref[...]
    u_ref[...] = jnp.dot(h, w1_ref[...], preferred_element_type=jnp.float32)


def _k0b_body(u_ref, d0_ref, d1_ref, g2_ref, dinv_ref):
    deg = d0_ref[0, 0, :] + d1_ref[0, 0, :]
    dinv = lax.rsqrt(deg)
    dinv_ref[0, 0, :] = dinv
    gs = u_ref[...] * dinv[:, None]
    g2_ref[0, :, :] = gs[:, :HH]
    g2_ref[1, :, :] = gs[:, HH:]


def _k12_body(aA_ref, aB_ref, dinv_ref, b_ref, w_ref, x_ref, g2_ref):
    dinv = dinv_ref[0, 0, :]
    a = jnp.concatenate([aA_ref[...], aB_ref[...]], axis=1)
    xl = jnp.maximum(a * dinv[:, None] + b_ref[...], 0.0)
    x_ref[...] = xl
    g = jnp.dot(xl, w_ref[...], preferred_element_type=jnp.float32) * dinv[:, None]
    g2_ref[0, :, :] = g[:, :HH]
    g2_ref[1, :, :] = g[:, HH:]


def _k3_body(aA_ref, aB_ref, dinv_ref, b3_ref, x1_ref, x2_ref, batch_ref,
             wp1_ref, bp1_ref, wp2_ref, bp2_ref, z_ref, pooled, cnt):
    i = pl.program_id(0)

    @pl.when(i == 0)
    def _():
        pooled[...] = jnp.zeros_like(pooled)
        cnt[...] = jnp.zeros_like(cnt)

    dinv = dinv_ref[0, 0, :]
    a = jnp.concatenate([aA_ref[...], aB_ref[...]], axis=1)
    x3 = jnp.maximum(a * dinv[:, None] + b3_ref[...], 0.0)
    xs = jnp.concatenate([x1_ref[...], x2_ref[...], x3], axis=1)
    b = batch_ref[0, 0, :]
    gid = lax.broadcasted_iota(jnp.int32, (G, BN), 0)
    mask = (gid == b[None, :]).astype(jnp.float32)
    pooled[...] += jnp.dot(mask, xs, preferred_element_type=jnp.float32)
    cnt[...] += jnp.sum(mask, axis=1)[:, None]

    @pl.when(i == pl.num_programs(0) - 1)
    def _():
        mean = pooled[...] / jnp.maximum(cnt[...], 1.0)
        z1 = jnp.dot(mean, wp1_ref[...], preferred_element_type=jnp.float32)
        z1 = jnp.maximum(z1 + bp1_ref[...], 0.0)
        z = jnp.dot(z1, wp2_ref[...], preferred_element_type=jnp.float32) + bp2_ref[...]
        nrm = jnp.sqrt(jnp.sum(z * z, axis=1, keepdims=True))
        z_ref[...] = z / jnp.maximum(nrm, 1e-12)


def _row_spec(cols):
    return pl.BlockSpec((BN, cols), lambda i: (i, 0))


def _full_spec(shape):
    nd = len(shape)
    return pl.BlockSpec(shape, lambda i: (0,) * nd)


_VEC_SPEC = pl.BlockSpec((1, 1, BN), lambda i: (i, 0, 0))


_G2_SPEC = pl.BlockSpec((2, BN, HH), lambda i: (0, i, 0))


def _k0a(x, Win, b_in, W1):
    return pl.pallas_call(
        _k0a_body,
        grid=(NB,),
        in_specs=[
            _row_spec(F_IN),
            _full_spec((F_IN, H)),
            _full_spec((1, H)),
            _full_spec((H, H)),
        ],
        out_specs=_row_spec(H),
        out_shape=jax.ShapeDtypeStruct((N, H), jnp.float32),
    )(x, Win, b_in, W1)


def _k0b(u, d0, d1):
    return pl.pallas_call(
        _k0b_body,
        grid=(NB,),
        in_specs=[_row_spec(H), _VEC_SPEC, _VEC_SPEC],
        out_specs=[_G2_SPEC, _VEC_SPEC],
        out_shape=[
            jax.ShapeDtypeStruct((2, N, HH), jnp.float32),
            jax.ShapeDtypeStruct((NB, 1, BN), jnp.float32),
        ],
    )(u, d0, d1)


def _k12(aA, aB, dinv, b, W):
    return pl.pallas_call(
        _k12_body,
        grid=(NB,),
        in_specs=[
            _row_spec(HH),
            _row_spec(HH),
            _VEC_SPEC,
            _full_spec((1, H)),
            _full_spec((H, H)),
        ],
        out_specs=[_row_spec(H), _G2_SPEC],
        out_shape=[
            jax.ShapeDtypeStruct((N, H), jnp.float32),
            jax.ShapeDtypeStruct((2, N, HH), jnp.float32),
        ],
    )(aA, aB, dinv, b, W)


def _k3(aA, aB, dinv, b3, x1, x2, batch3, Wp1, bp1, Wp2, bp2):
    return pl.pallas_call(
        _k3_body,
        grid=(NB,),
        in_specs=[
            _row_spec(HH),
            _row_spec(HH),
            _VEC_SPEC,
            _full_spec((1, H)),
            _row_spec(H),
            _row_spec(H),
            pl.BlockSpec((1, 1, BN), lambda i: (i, 0, 0)),
            _full_spec((3 * H, H)),
            _full_spec((1, H)),
            _full_spec((H, P)),
            _full_spec((1, P)),
        ],
        out_specs=pl.BlockSpec((G, P), lambda i: (0, 0)),
        out_shape=jax.ShapeDtypeStruct((G, P), jnp.float32),
        scratch_shapes=[
            pltpu.VMEM((G, 3 * H), jnp.float32),
            pltpu.VMEM((G, 1), jnp.float32),
        ],
    )(aA, aB, dinv, b3, x1, x2, batch3, Wp1, bp1, Wp2, bp2)


def kernel(x, edge_index, batch, W_in, b_in, W1, b1, W2, b2, W3, b3, Wp1, bp1, Wp2, bp2):
    src = edge_index[0]
    dst = edge_index[1]
    loop = jnp.arange(N, dtype=jnp.int32)
    pad = EP - E - N
    srcE = jnp.concatenate([src, loop, jnp.zeros((pad,), jnp.int32)])
    dstE = jnp.concatenate([dst, loop, jnp.full((pad,), DUMMY, jnp.int32)])
    dstE3 = dstE.reshape(NT, NCHUNK, CE)
    s5 = srcE.reshape(NT, NCH2, 2, 2, CE // 2)
    d5 = dstE.reshape(NT, NCH2, 2, 2, CE // 2)
    srcP = (s5[:, :, :, 0, :] | (s5[:, :, :, 1, :] << 16)).reshape(NT, NCH2, CE)
    dstP = (d5[:, :, :, 0, :] | (d5[:, :, :, 1, :] << 16)).reshape(NT, NCH2, CE)

    deg2 = _deg_call(dstE3)
    d0 = deg2[0, :N].reshape(NB, 1, BN)
    d1 = deg2[1, :N].reshape(NB, 1, BN)

    b_in2 = b_in.reshape(1, H)
    b1_2 = b1.reshape(1, H)
    b2_2 = b2.reshape(1, H)
    b3_2 = b3.reshape(1, H)
    bp1_2 = bp1.reshape(1, H)
    bp2_2 = bp2.reshape(1, P)
    batch3 = batch.reshape(NB, 1, BN)

    u = _k0a(x, W_in, b_in2, W1)
    g2, dinv = _k0b(u, d0, d1)
    o1 = _layer_call(g2.reshape(2 * N, HH), srcP, dstP)
    x1, g2b = _k12(o1[:N], o1[N:], dinv, b1_2, W2)
    o2 = _layer_call(g2b.reshape(2 * N, HH), srcP, dstP)
    x2, g2c = _k12(o2[:N], o2[N:], dinv, b2_2, W3)
    o3 = _layer_call(g2c.reshape(2 * N, HH), srcP, dstP)
    z = _k3(o3[:N], o3[N:], dinv, b3_2, x1, x2, batch3, Wp1, bp1_2, Wp2, bp2_2)
    return z


# speedup vs baseline: 11.5058x; 1.0629x over previous
import jax
import jax.numpy as jnp
from jax import lax
from jax.experimental import pallas as pl
from jax.experimental.pallas import tpu as pltpu
from jax.experimental.pallas import tpu_sc as plsc

N = 10000
E = 160000
F_IN = 256
H = 256
HH = 128
G = 128
P = 128

NT = 16
CE = 128
NCHUNK = 84
EP = NT * CE * NCHUNK
DUMMY = N
NPAD = 10240
WBR = 624
WBL = N - (NT - 1) * WBR

BN = 1000
NB = N // BN

def _mesh():
    return plsc.VectorSubcoreMesh(
        core_axis_name="c", subcore_axis_name="s", num_cores=2, num_subcores=NT
    )


def _sc_deg_body(dstE, dout, slots, didx, dloc, tmp, tmp2):
    cid = lax.axis_index("c")
    tid = lax.axis_index("s")
    zv = jnp.zeros((16,), jnp.float32)

    def _z(i, _):
        dloc[pl.ds(i * 16, 16)] = zv
        return 0

    lax.fori_loop(0, NPAD // 16, _z, 0)
    pltpu.sync_copy(dstE.at[tid], didx)
    ones = jnp.full((16,), 1.0, jnp.float32)
    half = NCHUNK // 2

    def _chunk(j, _):
        jj = cid * half + j

        def _k(k, _):
            idx = didx[jj, pl.ds(k * 16, 16)]
            plsc.addupdate_scatter(dloc, [idx], ones)
            return 0

        lax.fori_loop(0, CE // 16, _k, 0)
        return 0

    lax.fori_loop(0, half, _chunk, 0)
    pltpu.sync_copy(dloc, slots.at[tid])
    plsc.subcore_barrier()

    def _z2(i, _):
        tmp[pl.ds(i * 16, 16)] = zv
        return 0

    lax.fori_loop(0, 640 // 16, _z2, 0)

    def _s(s, _):
        pltpu.sync_copy(slots.at[s, pl.ds(pl.multiple_of(tid * 640, 128), 640)], tmp2)

        def _a(i, _):
            tmp[pl.ds(i * 16, 16)] = tmp[pl.ds(i * 16, 16)] + tmp2[pl.ds(i * 16, 16)]
            return 0

        lax.fori_loop(0, 640 // 16, _a, 0)
        return 0

    lax.fori_loop(0, NT, _s, 0)

    obase = pl.multiple_of(tid * 640, 128)

    @pl.when(cid == 0)
    def _():
        pltpu.sync_copy(tmp, dout.at[0, pl.ds(obase, 640)])

    @pl.when(cid == 1)
    def _():
        pltpu.sync_copy(tmp, dout.at[1, pl.ds(obase, 640)])


def _deg_call(dstE3):
    return pl.kernel(
        _sc_deg_body,
        out_type=jax.ShapeDtypeStruct((2, NPAD), jnp.float32),
        mesh=_mesh(),
        compiler_params=pltpu.CompilerParams(needs_layout_passes=False),
        scratch_types=[
            pltpu.VMEM_SHARED((NT, NPAD), jnp.float32),
            pltpu.VMEM((NCHUNK, CE), jnp.int32),
            pltpu.VMEM((NPAD,), jnp.float32),
            pltpu.VMEM((640,), jnp.float32),
            pltpu.VMEM((640,), jnp.float32),
        ],
    )(dstE3)


NCH2 = NCHUNK // 2


def _sc_layer_body(g, srcP, dstP, out,
                   acc, spk, dpk, sb0, sb1, db, gbuf0, gbuf1, sem0, sem1):
    cid = lax.axis_index("c")
    tid = lax.axis_index("s")
    zv = jnp.zeros((16,), jnp.float32)
    goff = jnp.full((16,), 0, jnp.int32) + cid * N

    def _zr(r, _):
        def _zc(k, _):
            gbuf0[r, pl.ds(k * 16, 16)] = zv
            return 0

        lax.fori_loop(0, HH // 16, _zc, 0)
        return 0

    lax.fori_loop(0, CE, _zr, 0)

    def _za(k, _):
        pltpu.sync_copy(gbuf0, acc.at[pl.ds(pl.multiple_of(tid * 640 + k * 128, 128), 128)])
        return 0

    lax.fori_loop(0, 5, _za, 0)
    pltpu.sync_copy(srcP.at[tid], spk)
    pltpu.sync_copy(dstP.at[tid], dpk)
    plsc.subcore_barrier()

    def _unpack_src(m, off, buf):
        for k in range(4):
            w = spk[m, pl.ds(off + k * 16, 16)]
            buf[pl.ds(k * 16, 16)] = (w & 0xFFFF) + goff
            buf[pl.ds(64 + k * 16, 16)] = lax.shift_right_logical(w, 16) + goff

    def _unpack_dst(m, off, buf):
        for k in range(4):
            w = dpk[m, pl.ds(off + k * 16, 16)]
            buf[pl.ds(k * 16, 16)] = w & 0xFFFF
            buf[pl.ds(64 + k * 16, 16)] = lax.shift_right_logical(w, 16)

    _unpack_src(0, 0, sb0)
    pltpu.async_copy(g.at[sb0], gbuf0, sem0)

    def _pair(j2, _):
        _unpack_src(j2, 64, sb1)
        pltpu.async_copy(g.at[sb1], gbuf1, sem1)
        _unpack_dst(j2, 0, db)
        pltpu.make_async_copy(g.at[sb0], gbuf0, sem0).wait()
        pltpu.sync_copy(gbuf0, acc.at[db], add=True)

        @pl.when(j2 < NCH2 - 1)
        def _():
            _unpack_src(j2 + 1, 0, sb0)
            pltpu.async_copy(g.at[sb0], gbuf0, sem0)

        _unpack_dst(j2, 64, db)
        pltpu.make_async_copy(g.at[sb1], gbuf1, sem1).wait()
        pltpu.sync_copy(gbuf1, acc.at[db], add=True)
        return 0

    lax.fori_loop(0, NCH2, _pair, 0)
    plsc.subcore_barrier()
    base = pl.multiple_of(tid * WBR, 16)
    obase = pl.multiple_of(cid * N + tid * WBR, 16)

    @pl.when(tid < NT - 1)
    def _():
        pltpu.sync_copy(acc.at[pl.ds(base, WBR)], out.at[pl.ds(obase, WBR)])

    @pl.when(tid == NT - 1)
    def _():
        pltpu.sync_copy(acc.at[pl.ds(base, WBL)], out.at[pl.ds(obase, WBL)])


def _layer_call(g2, srcP, dstP):
    return pl.kernel(
        _sc_layer_body,
        out_type=jax.ShapeDtypeStruct((2 * N, HH), jnp.float32),
        mesh=_mesh(),
        scratch_types=[
            pltpu.VMEM_SHARED((NPAD, HH), jnp.float32),
            pltpu.VMEM((NCH2, CE), jnp.int32),
            pltpu.VMEM((NCH2, CE), jnp.int32),
            pltpu.VMEM((CE,), jnp.int32),
            pltpu.VMEM((CE,), jnp.int32),
            pltpu.VMEM((CE,), jnp.int32),
            pltpu.VMEM((CE, HH), jnp.float32),
            pltpu.VMEM((CE, HH), jnp.float32),
            pltpu.SemaphoreType.DMA,
            pltpu.SemaphoreType.DMA,
        ],
    )(g2, srcP, dstP)


def _k0a_body(x_ref, win_ref, bin_ref, w1_ref, u_ref):
    h = jnp.dot(x_ref[...], win_ref[...], preferred_element_type=jnp.float32)
    h = h + bin_ref[...]
    u_ref[...] = jnp.dot(h, w1_ref[...], preferred_element_type=jnp.float32)


def _k0b_body(u_ref, d0_ref, d1_ref, g2_ref, dinv_ref):
    deg = d0_ref[0, 0, :] + d1_ref[0, 0, :]
    dinv = lax.rsqrt(deg)
    dinv_ref[0, 0, :] = dinv
    gs = u_ref[...] * dinv[:, None]
    g2_ref[0, :, :] = gs[:, :HH]
    g2_ref[1, :, :] = gs[:, HH:]


def _k12_body(aA_ref, aB_ref, dinv_ref, b_ref, w_ref, x_ref, g2_ref):
    dinv = dinv_ref[0, 0, :]
    a = jnp.concatenate([aA_ref[...], aB_ref[...]], axis=1).astype(jnp.float32)
    xl = jnp.maximum(a * dinv[:, None] + b_ref[...], 0.0)
    x_ref[...] = xl
    gs = jnp.dot(xl, w_ref[...], preferred_element_type=jnp.float32) * dinv[:, None]
    g2_ref[0, :, :] = gs[:, :HH]
    g2_ref[1, :, :] = gs[:, HH:]


def _k3_body(aA_ref, aB_ref, dinv_ref, b3_ref, x1_ref, x2_ref, batch_ref,
             wp1_ref, bp1_ref, wp2_ref, bp2_ref, z_ref, pooled, cnt):
    i = pl.program_id(0)

    @pl.when(i == 0)
    def _():
        pooled[...] = jnp.zeros_like(pooled)
        cnt[...] = jnp.zeros_like(cnt)

    dinv = dinv_ref[0, 0, :]
    a = jnp.concatenate([aA_ref[...], aB_ref[...]], axis=1).astype(jnp.float32)
    x3 = jnp.maximum(a * dinv[:, None] + b3_ref[...], 0.0)
    xs = jnp.concatenate([x1_ref[...], x2_ref[...], x3], axis=1)
    b = batch_ref[0, 0, :]
    gid = lax.broadcasted_iota(jnp.int32, (G, BN), 0)
    mask = (gid == b[None, :]).astype(jnp.float32)
    pooled[...] += jnp.dot(mask, xs, preferred_element_type=jnp.float32)
    cnt[...] += jnp.sum(mask, axis=1)[:, None]

    @pl.when(i == pl.num_programs(0) - 1)
    def _():
        mean = pooled[...] / jnp.maximum(cnt[...], 1.0)
        z1 = jnp.dot(mean, wp1_ref[...], preferred_element_type=jnp.float32)
        z1 = jnp.maximum(z1 + bp1_ref[...], 0.0)
        z = jnp.dot(z1, wp2_ref[...], preferred_element_type=jnp.float32) + bp2_ref[...]
        nrm = jnp.sqrt(jnp.sum(z * z, axis=1, keepdims=True))
        z_ref[...] = z / jnp.maximum(nrm, 1e-12)


def _row_spec(cols):
    return pl.BlockSpec((BN, cols), lambda i: (i, 0))


def _full_spec(shape):
    nd = len(shape)
    return pl.BlockSpec(shape, lambda i: (0,) * nd)


_VEC_SPEC = pl.BlockSpec((1, 1, BN), lambda i: (i, 0, 0))


_G2_SPEC = pl.BlockSpec((2, BN, HH), lambda i: (0, i, 0))


def _k0a(x, Win, b_in, W1):
    return pl.pallas_call(
        _k0a_body,
        grid=(NB,),
        in_specs=[
            _row_spec(F_IN),
            _full_spec((F_IN, H)),
            _full_spec((1, H)),
            _full_spec((H, H)),
        ],
        out_specs=_row_spec(H),
        out_shape=jax.ShapeDtypeStruct((N, H), jnp.float32),
    )(x, Win, b_in, W1)


def _k0b(u, d0, d1):
    return pl.pallas_call(
        _k0b_body,
        grid=(NB,),
        in_specs=[_row_spec(H), _VEC_SPEC, _VEC_SPEC],
        out_specs=[_G2_SPEC, _VEC_SPEC],
        out_shape=[
            jax.ShapeDtypeStruct((2, N, HH), jnp.float32),
            jax.ShapeDtypeStruct((NB, 1, BN), jnp.float32),
        ],
    )(u, d0, d1)


_ACC_A_SPEC = pl.BlockSpec((BN, HH), lambda i: (i, 0))
_ACC_B_SPEC = pl.BlockSpec((BN, HH), lambda i: (i + NB, 0))


def _k12(aA, aB, dinv, b, W):
    return pl.pallas_call(
        _k12_body,
        grid=(NB,),
        in_specs=[
            _ACC_A_SPEC,
            _ACC_B_SPEC,
            _VEC_SPEC,
            _full_spec((1, H)),
            _full_spec((H, H)),
        ],
        out_specs=[_row_spec(H), _G2_SPEC],
        out_shape=[
            jax.ShapeDtypeStruct((N, H), jnp.float32),
            jax.ShapeDtypeStruct((2, N, HH), jnp.float32),
        ],
    )(aA, aB, dinv, b, W)


def _k3(aA, aB, dinv, b3, x1, x2, batch3, Wp1, bp1, Wp2, bp2):
    return pl.pallas_call(
        _k3_body,
        grid=(NB,),
        in_specs=[
            _ACC_A_SPEC,
            _ACC_B_SPEC,
            _VEC_SPEC,
            _full_spec((1, H)),
            _row_spec(H),
            _row_spec(H),
            pl.BlockSpec((1, 1, BN), lambda i: (i, 0, 0)),
            _full_spec((3 * H, H)),
            _full_spec((1, H)),
            _full_spec((H, P)),
            _full_spec((1, P)),
        ],
        out_specs=pl.BlockSpec((G, P), lambda i: (0, 0)),
        out_shape=jax.ShapeDtypeStruct((G, P), jnp.float32),
        scratch_shapes=[
            pltpu.VMEM((G, 3 * H), jnp.float32),
            pltpu.VMEM((G, 1), jnp.float32),
        ],
    )(aA, aB, dinv, b3, x1, x2, batch3, Wp1, bp1, Wp2, bp2)


def kernel(x, edge_index, batch, W_in, b_in, W1, b1, W2, b2, W3, b3, Wp1, bp1, Wp2, bp2):
    src = edge_index[0]
    dst = edge_index[1]
    loop = jnp.arange(N, dtype=jnp.int32)
    pad = EP - E - N
    srcE = jnp.concatenate([src, loop, jnp.zeros((pad,), jnp.int32)])
    dstE = jnp.concatenate([dst, loop, jnp.full((pad,), DUMMY, jnp.int32)])
    dstE3 = dstE.reshape(NT, NCHUNK, CE)
    s5 = srcE.reshape(NT, NCH2, 2, 2, CE // 2)
    d5 = dstE.reshape(NT, NCH2, 2, 2, CE // 2)
    srcP = (s5[:, :, :, 0, :] | (s5[:, :, :, 1, :] << 16)).reshape(NT, NCH2, CE)
    dstP = (d5[:, :, :, 0, :] | (d5[:, :, :, 1, :] << 16)).reshape(NT, NCH2, CE)

    deg2 = _deg_call(dstE3)
    d0 = deg2[0, :N].reshape(NB, 1, BN)
    d1 = deg2[1, :N].reshape(NB, 1, BN)

    b_in2 = b_in.reshape(1, H)
    b1_2 = b1.reshape(1, H)
    b2_2 = b2.reshape(1, H)
    b3_2 = b3.reshape(1, H)
    bp1_2 = bp1.reshape(1, H)
    bp2_2 = bp2.reshape(1, P)
    batch3 = batch.reshape(NB, 1, BN)

    u = _k0a(x, W_in, b_in2, W1)
    g2, dinv = _k0b(u, d0, d1)
    o1 = _layer_call(g2.reshape(2 * N, HH), srcP, dstP)
    x1, g2b = _k12(o1, o1, dinv, b1_2, W2)
    o2 = _layer_call(g2b.reshape(2 * N, HH), srcP, dstP)
    x2, g2c = _k12(o2, o2, dinv, b2_2, W3)
    o3 = _layer_call(g2c.reshape(2 * N, HH), srcP, dstP)
    z = _k3(o3, o3, dinv, b3_2, x1, x2, batch3, Wp1, bp1_2, Wp2, bp2_2)
    return z
